# Initial kernel scaffold; baseline (speedup 1.0000x reference)
import jax
import jax.numpy as jnp
from jax.experimental import pallas as pl


def kernel(x, edge_index, degrees, W1, b1, W2, b2):
    raise NotImplementedError("write your pallas kernel here")


# speedup vs baseline: 5.5633x; 5.5633x over previous
import functools

import jax
import jax.numpy as jnp
from jax import lax
from jax.experimental import pallas as pl
from jax.experimental.pallas import tpu as pltpu
from jax.experimental.pallas import tpu_sc as plsc

N_NODES = 10000
D = 128
DH = D // 2
N_EDGES = 320000

NC = 2
NS = 16
E_PER_S = N_EDGES // NS
K = 80
NCHUNK = E_PER_S // K
N_PAD = 10240
ROWS_PER_S = N_PAD // NS
ZROWS = 128

_mesh = plsc.VectorSubcoreMesh(core_axis_name="c", subcore_axis_name="s")


@functools.partial(
    pl.kernel,
    out_type=jax.ShapeDtypeStruct((NC, N_PAD, DH), jnp.float32),
    mesh=_mesh,
    scratch_types=[
        pltpu.VMEM((NCHUNK, K), jnp.int32),
        pltpu.VMEM((NCHUNK, K), jnp.int32),
        pltpu.VMEM((K, DH), jnp.float32),
        pltpu.VMEM((ZROWS, DH), jnp.float32),
        pltpu.VMEM_SHARED((N_PAD, DH), jnp.float32),
        pltpu.SemaphoreType.DMA,
    ],
    compiler_params=pltpu.CompilerParams(use_tc_tiling_on_sc=False),
)
def _aggregate(src_hbm, dst_hbm, xl_hbm, xr_hbm, part_hbm,
               src_v, dst_v, rows_v, zbuf, acc, sem):
    c = lax.axis_index("c")
    s = lax.axis_index("s")

    pltpu.sync_copy(src_hbm.at[s], src_v)
    pltpu.sync_copy(dst_hbm.at[s], dst_v)

    def zrow(i, carry):
        def zcol(j, carry2):
            zbuf[i, pl.ds(j * 16, 16)] = jnp.zeros((16,), jnp.float32)
            return carry2
        return lax.fori_loop(0, DH // 16, zcol, carry)
    lax.fori_loop(0, ZROWS, zrow, 0)
    for r in range(ROWS_PER_S // ZROWS):
        pltpu.sync_copy(zbuf, acc.at[pl.ds(s * ROWS_PER_S + r * ZROWS, ZROWS)])

    plsc.subcore_barrier()

    def run(x_tab):
        def body(i, carry):
            pltpu.async_copy(x_tab.at[src_v.at[i]], rows_v, sem).wait()
            pltpu.sync_copy(rows_v, acc.at[dst_v.at[i]], add=True)
            return carry
        lax.fori_loop(0, NCHUNK, body, 0)

    pl.when(c == 0)(lambda: run(xl_hbm))
    pl.when(c == 1)(lambda: run(xr_hbm))

    plsc.subcore_barrier()

    pltpu.sync_copy(acc.at[pl.ds(s * ROWS_PER_S, ROWS_PER_S)],
                    part_hbm.at[c, pl.ds(s * ROWS_PER_S, ROWS_PER_S)])


BLK = 1000


def _mlp_body(x_ref, p0_ref, p1_ref, w1_ref, b1_ref, w2_ref, b2_ref, o_ref):
    msg = jnp.concatenate([p0_ref[...], p1_ref[...]], axis=1)
    h = x_ref[...] + msg
    h1 = jnp.dot(h, w1_ref[...], preferred_element_type=jnp.float32)
    h1 = jnp.maximum(h1 + b1_ref[...], 0.0)
    o_ref[...] = jnp.dot(h1, w2_ref[...],
                         preferred_element_type=jnp.float32) + b2_ref[...]


_mlp = pl.pallas_call(
    _mlp_body,
    out_shape=jax.ShapeDtypeStruct((N_NODES, D), jnp.float32),
    grid=(N_NODES // BLK,),
    in_specs=[
        pl.BlockSpec((BLK, D), lambda i: (i, 0)),
        pl.BlockSpec((BLK, DH), lambda i: (i, 0)),
        pl.BlockSpec((BLK, DH), lambda i: (i, 0)),
        pl.BlockSpec((D, D), lambda i: (0, 0)),
        pl.BlockSpec((1, D), lambda i: (0, 0)),
        pl.BlockSpec((D, D), lambda i: (0, 0)),
        pl.BlockSpec((1, D), lambda i: (0, 0)),
    ],
    out_specs=pl.BlockSpec((BLK, D), lambda i: (i, 0)),
)


def kernel(x, edge_index, degrees, W1, b1, W2, b2):
    src = edge_index[0].astype(jnp.int32).reshape(NS, NCHUNK, K)
    dst = edge_index[1].astype(jnp.int32).reshape(NS, NCHUNK, K)
    part = _aggregate(src, dst, x[:, :DH], x[:, DH:])
    return _mlp(x, part[0], part[1], W1, b1.reshape(1, D), W2, b2.reshape(1, D))


# speedup vs baseline: 6.7339x; 1.2104x over previous
import functools

import jax
import jax.numpy as jnp
from jax import lax
from jax.experimental import pallas as pl
from jax.experimental.pallas import tpu as pltpu
from jax.experimental.pallas import tpu_sc as plsc

N_NODES = 10000
D = 128
DH = D // 2
N_EDGES = 320000

NC = 2
NS = 16
E_PER_S = N_EDGES // NS
K = 80
NCHUNK = E_PER_S // K
N_PAD = 10240
ROWS_PER_S = N_PAD // NS
ZROWS = 128

_mesh = plsc.VectorSubcoreMesh(core_axis_name="c", subcore_axis_name="s")


@functools.partial(
    pl.kernel,
    out_type=jax.ShapeDtypeStruct((NC, N_PAD, DH), jnp.float32),
    mesh=_mesh,
    scratch_types=[
        pltpu.VMEM((NCHUNK, K), jnp.int32),
        pltpu.VMEM((NCHUNK, K), jnp.int32),
        pltpu.VMEM((K, DH), jnp.float32),
        pltpu.VMEM((K, DH), jnp.float32),
        pltpu.VMEM((ZROWS, DH), jnp.float32),
        pltpu.VMEM_SHARED((N_PAD, DH), jnp.float32),
        pltpu.SemaphoreType.DMA,
    ],
    compiler_params=pltpu.CompilerParams(use_tc_tiling_on_sc=False),
)
def _aggregate(src_hbm, dst_hbm, xl_hbm, xr_hbm, part_hbm,
               src_v, dst_v, rows0, rows1, zbuf, acc, sem):
    c = lax.axis_index("c")
    s = lax.axis_index("s")

    pltpu.sync_copy(src_hbm.at[s], src_v)
    pltpu.sync_copy(dst_hbm.at[s], dst_v)

    def zrow(i, carry):
        def zcol(j, carry2):
            zbuf[i, pl.ds(j * 16, 16)] = jnp.zeros((16,), jnp.float32)
            return carry2
        return lax.fori_loop(0, DH // 16, zcol, carry)
    lax.fori_loop(0, ZROWS, zrow, 0)
    for r in range(ROWS_PER_S // ZROWS):
        pltpu.sync_copy(zbuf, acc.at[pl.ds(s * ROWS_PER_S + r * ZROWS, ZROWS)])

    plsc.subcore_barrier()

    def run(x_tab):
        def issue(i, buf):
            pltpu.async_copy(x_tab.at[src_v.at[i]], buf, sem)

        def wait(i, buf):
            pltpu.make_async_copy(x_tab.at[src_v.at[i]], buf, sem).wait()

        def scat(i, buf):
            pltpu.sync_copy(buf, acc.at[dst_v.at[i]], add=True)

        issue(0, rows0)

        def body(g, carry):
            i0 = 2 * g
            wait(i0, rows0)
            issue(i0 + 1, rows1)
            scat(i0, rows0)
            wait(i0 + 1, rows1)
            issue(i0 + 2, rows0)
            scat(i0 + 1, rows1)
            return carry
        lax.fori_loop(0, NCHUNK // 2 - 1, body, 0)

        i0 = NCHUNK - 2
        wait(i0, rows0)
        issue(i0 + 1, rows1)
        scat(i0, rows0)
        wait(i0 + 1, rows1)
        scat(i0 + 1, rows1)

    pl.when(c == 0)(lambda: run(xl_hbm))
    pl.when(c == 1)(lambda: run(xr_hbm))

    plsc.subcore_barrier()

    pltpu.sync_copy(acc.at[pl.ds(s * ROWS_PER_S, ROWS_PER_S)],
                    part_hbm.at[c, pl.ds(s * ROWS_PER_S, ROWS_PER_S)])


BLK = 1000


def _mlp_body(x_ref, p0_ref, p1_ref, w1_ref, b1_ref, w2_ref, b2_ref, o_ref):
    msg = jnp.concatenate([p0_ref[...], p1_ref[...]], axis=1)
    h = x_ref[...] + msg
    h1 = jnp.dot(h, w1_ref[...], preferred_element_type=jnp.float32)
    h1 = jnp.maximum(h1 + b1_ref[...], 0.0)
    o_ref[...] = jnp.dot(h1, w2_ref[...],
                         preferred_element_type=jnp.float32) + b2_ref[...]


_mlp = pl.pallas_call(
    _mlp_body,
    out_shape=jax.ShapeDtypeStruct((N_NODES, D), jnp.float32),
    grid=(N_NODES // BLK,),
    in_specs=[
        pl.BlockSpec((BLK, D), lambda i: (i, 0)),
        pl.BlockSpec((BLK, DH), lambda i: (i, 0)),
        pl.BlockSpec((BLK, DH), lambda i: (i, 0)),
        pl.BlockSpec((D, D), lambda i: (0, 0)),
        pl.BlockSpec((1, D), lambda i: (0, 0)),
        pl.BlockSpec((D, D), lambda i: (0, 0)),
        pl.BlockSpec((1, D), lambda i: (0, 0)),
    ],
    out_specs=pl.BlockSpec((BLK, D), lambda i: (i, 0)),
)


def kernel(x, edge_index, degrees, W1, b1, W2, b2):
    src = edge_index[0].astype(jnp.int32).reshape(NS, NCHUNK, K)
    dst = edge_index[1].astype(jnp.int32).reshape(NS, NCHUNK, K)
    part = _aggregate(src, dst, x[:, :DH], x[:, DH:])
    return _mlp(x, part[0], part[1], W1, b1.reshape(1, D), W2, b2.reshape(1, D))
